# baseline (speedup 1.0000x reference)
import functools

import jax
import jax.numpy as jnp
from jax import lax
from jax.experimental import pallas as pl
from jax.experimental.pallas import tpu as pltpu
from jax.experimental.pallas import tpu_sc as plsc

INPUT_DIM = 1000
OUTPUT_DIM = 64
ROW = OUTPUT_DIM * OUTPUT_DIM
BATCH = 4096

NUM_CORES = 2
NUM_SUBCORES = 16
NUM_WORKERS = NUM_CORES * NUM_SUBCORES
CHUNK = 8

NPART = 4
PART = BATCH // NPART


def _build(batch_part):
  b_per_w = batch_part // NUM_WORKERS
  nchunk = b_per_w // CHUNK
  mesh = plsc.VectorSubcoreMesh(core_axis_name="c", subcore_axis_name="s")

  @functools.partial(
      pl.kernel,
      mesh=mesh,
      out_type=jax.ShapeDtypeStruct((batch_part, ROW), jnp.float32),
      scratch_types=[
          pltpu.VMEM((b_per_w,), jnp.int32),
          pltpu.VMEM((CHUNK, ROW), jnp.float32),
          pltpu.VMEM((CHUNK, ROW), jnp.float32),
          pltpu.SemaphoreType.DMA,
          pltpu.SemaphoreType.DMA,
          pltpu.SemaphoreType.DMA,
          pltpu.SemaphoreType.DMA,
      ],
  )
  def gather_kernel(idx_hbm, table_hbm, out_hbm, idx_v, buf0, buf1,
                    gsem0, gsem1, ssem0, ssem1):
    wid = lax.axis_index("s") * NUM_CORES + lax.axis_index("c")
    base = wid * b_per_w
    pltpu.sync_copy(idx_hbm.at[pl.ds(base, b_per_w)], idx_v)

    bufs = (buf0, buf1)
    gsems = (gsem0, gsem1)
    ssems = (ssem0, ssem1)

    def gather(g, b):
      return pltpu.async_copy(
          table_hbm.at[idx_v.at[pl.ds(g * CHUNK, CHUNK)]], bufs[b], gsems[b])

    def scatter(g, b):
      return pltpu.async_copy(
          bufs[b], out_hbm.at[pl.ds(base + g * CHUNK, CHUNK)], ssems[b])

    gd = [None] * nchunk
    sd = [None] * nchunk
    gd[0] = gather(0, 0)
    for g in range(nchunk):
      b = g % 2
      gd[g].wait()
      sd[g] = scatter(g, b)
      if g + 1 < nchunk:
        if g >= 1:
          sd[g - 1].wait()
        gd[g + 1] = gather(g + 1, 1 - b)
    if nchunk >= 2:
      sd[nchunk - 2].wait()
    sd[nchunk - 1].wait()

  return gather_kernel


_gather_part = _build(PART)


def kernel(inputs, embeddings):
  table = embeddings.reshape(INPUT_DIM, ROW)
  parts = []
  for k in range(NPART):
    flat = _gather_part(lax.slice(inputs, (k * PART,), ((k + 1) * PART,)),
                        table)
    parts.append(flat.reshape(PART, OUTPUT_DIM, OUTPUT_DIM))
  return jnp.concatenate(parts, axis=0)
